# baseline (speedup 1.0000x reference)
import functools

import jax
import jax.numpy as jnp
from jax import lax
from jax.experimental import pallas as pl
from jax.experimental.pallas import tpu as pltpu
from jax.experimental.pallas import tpu_sc as plsc

N_NODES = 10000
N_EDGES = 320000
D_H = 128
N_CLS = 25
CP = 32

NC = 2
NS = 16
NW = NC * NS
K = 112
SLOW_CORE = 0
CH_SLOW = 60
CH_FAST = 120
E_SLOW = NS * CH_SLOW * K
E_PAD = NS * (CH_SLOW + CH_FAST) * K
NPAD = 10112
ZROWS = NPAD // NS

_MESH = plsc.VectorSubcoreMesh(core_axis_name="c", subcore_axis_name="s")


@functools.partial(
    pl.kernel,
    out_type=jax.ShapeDtypeStruct((NC, NPAD, D_H), jnp.float32),
    mesh=_MESH,
    compiler_params=pltpu.CompilerParams(needs_layout_passes=False),
    scratch_types=[
        pltpu.VMEM((CH_FAST, K), jnp.int32),
        pltpu.VMEM((8, K), jnp.int32),
        pltpu.VMEM((8, K), jnp.int32),
        pltpu.VMEM((K, D_H), jnp.float32),
        pltpu.VMEM((K, D_H), jnp.float32),
        pltpu.VMEM_SHARED((NPAD, D_H), jnp.float32),
        pltpu.SemaphoreType.DMA,
        pltpu.SemaphoreType.DMA,
    ],
)
def _sc_scatter(h_hbm, pk0_hbm, pk1_hbm, out_hbm, pk_v, usrc, udst,
                rows_a, rows_b, acc, sem_a, sem_b):
    c = lax.axis_index("c")
    s = lax.axis_index("s")

    @pl.when(c == SLOW_CORE)
    def _():
        pltpu.sync_copy(pk0_hbm.at[s], pk_v.at[pl.ds(0, CH_SLOW)])

    @pl.when(c != SLOW_CORE)
    def _():
        pltpu.sync_copy(pk1_hbm.at[s], pk_v)

    def unpack(j, r):
        for t in range(K // 16):
            w = pk_v[j, pl.ds(t * 16, 16)]
            usrc[r, pl.ds(t * 16, 16)] = w & 0xFFFF
            udst[r, pl.ds(t * 16, 16)] = lax.shift_right_logical(w, 16)

    def zbody(i, carry):
        rows_b[i // (D_H // 16), pl.ds((i % (D_H // 16)) * 16, 16)] = (
            jnp.zeros((16,), jnp.float32))
        return carry

    lax.fori_loop(0, K * D_H // 16, zbody, 0)

    unpack(0, 0)
    pltpu.async_copy(h_hbm.at[usrc.at[0]], rows_a, sem_a)
    for z in range(0, ZROWS, K):
        zn = min(K, ZROWS - z)
        pltpu.sync_copy(rows_b.at[pl.ds(0, zn)],
                        acc.at[pl.ds(s * ZROWS + z, zn)])
    plsc.subcore_barrier()

    def make_body(ch):
        def body(i, carry):
            j = 2 * i
            unpack(j + 1, 1)
            pltpu.async_copy(h_hbm.at[usrc.at[1]], rows_b, sem_b)
            pltpu.make_async_copy(h_hbm.at[usrc.at[0]], rows_a, sem_a).wait()
            pltpu.sync_copy(rows_a, acc.at[udst.at[0]], add=True)

            @pl.when(j + 2 < ch)
            def _():
                unpack(j + 2, 0)
                pltpu.async_copy(h_hbm.at[usrc.at[0]], rows_a, sem_a)

            pltpu.make_async_copy(h_hbm.at[usrc.at[1]], rows_b, sem_b).wait()
            pltpu.sync_copy(rows_b, acc.at[udst.at[1]], add=True)
            return carry

        return body

    @pl.when(c == SLOW_CORE)
    def _():
        lax.fori_loop(0, CH_SLOW // 2, make_body(CH_SLOW), 0)

    @pl.when(c != SLOW_CORE)
    def _():
        lax.fori_loop(0, CH_FAST // 2, make_body(CH_FAST), 0)

    plsc.subcore_barrier()
    pltpu.sync_copy(acc.at[pl.ds(s * ZROWS, ZROWS)],
                    out_hbm.at[c].at[pl.ds(s * ZROWS, ZROWS)])


@functools.partial(
    pl.kernel,
    out_type=jax.ShapeDtypeStruct((NW, 1, NPAD), jnp.float32),
    mesh=_MESH,
    compiler_params=pltpu.CompilerParams(needs_layout_passes=False),
    scratch_types=[
        pltpu.VMEM((CH_FAST, K), jnp.int32),
        pltpu.VMEM((NPAD,), jnp.float32),
        pltpu.SemaphoreType.DMA,
    ],
)
def _sc_degree(pk0_hbm, pk1_hbm, out_hbm, pk_v, acc_v, sem):
    c = lax.axis_index("c")
    s = lax.axis_index("s")
    wid = s * NC + c

    @pl.when(c == SLOW_CORE)
    def _():
        pltpu.sync_copy(pk0_hbm.at[s], pk_v.at[pl.ds(0, CH_SLOW)])

    @pl.when(c != SLOW_CORE)
    def _():
        pltpu.sync_copy(pk1_hbm.at[s], pk_v)

    def zbody(i, carry):
        acc_v[pl.ds(i * 16, 16)] = jnp.zeros((16,), jnp.float32)
        return carry

    lax.fori_loop(0, NPAD // 16, zbody, 0)
    ones = jnp.ones((16,), jnp.float32)
    g = K // 16

    def body(i, carry):
        idx = lax.shift_right_logical(
            pk_v[i // g, pl.ds((i % g) * 16, 16)], 16)
        plsc.addupdate_scatter(acc_v, [idx], ones)
        return carry

    @pl.when(c == SLOW_CORE)
    def _():
        lax.fori_loop(0, CH_SLOW * g, body, 0)

    @pl.when(c != SLOW_CORE)
    def _():
        lax.fori_loop(0, CH_FAST * g, body, 0)

    pltpu.sync_copy(acc_v, out_hbm.at[wid].at[0])


_BM = 1024
_GRID = (N_NODES + _BM - 1) // _BM
_T_DN = (((0,), (0,)), ((), ()))


def _tc_first(x, W, degp):
    def body(x_ref, w_ref, degp_ref, h_ref, dinv_ref):
        colsum = lax.dot_general(degp_ref[...], jnp.ones((NW, 1), jnp.float32),
                                 _T_DN, preferred_element_type=jnp.float32)
        dinv = lax.rsqrt(1.0 + colsum)
        dinv_ref[...] = dinv
        h_ref[...] = jnp.dot(x_ref[...], w_ref[...],
                             preferred_element_type=jnp.float32) * dinv

    return pl.pallas_call(
        body,
        grid=(_GRID,),
        in_specs=[
            pl.BlockSpec((_BM, D_H), lambda i: (i, 0)),
            pl.BlockSpec((D_H, D_H), lambda i: (0, 0)),
            pl.BlockSpec((NW, _BM), lambda i: (0, i)),
        ],
        out_specs=[
            pl.BlockSpec((_BM, D_H), lambda i: (i, 0)),
            pl.BlockSpec((_BM, 1), lambda i: (i, 0)),
        ],
        out_shape=[
            jax.ShapeDtypeStruct((N_NODES, D_H), jnp.float32),
            jax.ShapeDtypeStruct((N_NODES, 1), jnp.float32),
        ],
    )(x, W, degp)


def _tc_mid(p, hp, dinv, b, W, relu=False):
    def body(p_ref, hp_ref, dinv_ref, b_ref, *rest):
        o_ref = rest[-1]
        dinv = dinv_ref[...]
        t = (p_ref[0] + p_ref[1] + hp_ref[...]) * dinv + b_ref[...]
        if relu:
            t = jnp.maximum(t, 0.0)
        if W is not None:
            t = jnp.dot(t, rest[0][...], preferred_element_type=jnp.float32)
        o_ref[...] = t * dinv

    in_specs = [
        pl.BlockSpec((NC, _BM, D_H), lambda i: (0, i, 0)),
        pl.BlockSpec((_BM, D_H), lambda i: (i, 0)),
        pl.BlockSpec((_BM, 1), lambda i: (i, 0)),
        pl.BlockSpec((1, D_H), lambda i: (0, 0)),
    ]
    args = [p, hp, dinv, b]
    if W is not None:
        in_specs.append(pl.BlockSpec((D_H, D_H), lambda i: (0, 0)))
        args.append(W)
    return pl.pallas_call(
        body,
        grid=(_GRID,),
        in_specs=in_specs,
        out_specs=pl.BlockSpec((_BM, D_H), lambda i: (i, 0)),
        out_shape=jax.ShapeDtypeStruct((N_NODES, D_H), jnp.float32),
    )(*args)


def _tc_final(p, u, dinv, W5, b5, Wl, bl):
    def body(p_ref, u_ref, dinv_ref, w5_ref, b5_ref, wl_ref, bl_ref, o_ref):
        t = jnp.dot(p_ref[0] + p_ref[1] + u_ref[...], w5_ref[...],
                    preferred_element_type=jnp.float32)
        out5 = t * dinv_ref[...] + b5_ref[...]
        o_ref[...] = jnp.dot(out5, wl_ref[...],
                             preferred_element_type=jnp.float32) + bl_ref[...]

    return pl.pallas_call(
        body,
        grid=(_GRID,),
        in_specs=[
            pl.BlockSpec((NC, _BM, D_H), lambda i: (0, i, 0)),
            pl.BlockSpec((_BM, D_H), lambda i: (i, 0)),
            pl.BlockSpec((_BM, 1), lambda i: (i, 0)),
            pl.BlockSpec((D_H, CP), lambda i: (0, 0)),
            pl.BlockSpec((1, CP), lambda i: (0, 0)),
            pl.BlockSpec((CP, CP), lambda i: (0, 0)),
            pl.BlockSpec((1, CP), lambda i: (0, 0)),
        ],
        out_specs=pl.BlockSpec((_BM, CP), lambda i: (i, 0)),
        out_shape=jax.ShapeDtypeStruct((N_NODES, CP), jnp.float32),
    )(p, u, dinv, W5, b5, Wl, bl)


def kernel(x, edge_index, W1, b1, W2, b2, W3, b3, W4, b4, W5, b5, Wl, bl):
    src = edge_index[0].astype(jnp.int32)
    dst = edge_index[1].astype(jnp.int32)
    pad = E_PAD - N_EDGES
    src_p = jnp.concatenate([src, jnp.zeros((pad,), jnp.int32)])
    trash = N_NODES + (jnp.arange(pad, dtype=jnp.int32) % (NPAD - N_NODES))
    dst_p = jnp.concatenate([dst, trash])
    pk_all = src_p | (dst_p << 16)
    pk0 = pk_all[:E_SLOW].reshape(NS, CH_SLOW, K)
    pk1 = pk_all[E_SLOW:].reshape(NS, CH_FAST, K)
    degp = _sc_degree(pk0, pk1).reshape(NW, NPAD)
    h1, dinv = _tc_first(x, W1, degp)
    p1 = _sc_scatter(h1, pk0, pk1)
    h2 = _tc_mid(p1, h1, dinv, b1.reshape(1, D_H), W2, relu=True)
    p2 = _sc_scatter(h2, pk0, pk1)
    h3 = _tc_mid(p2, h2, dinv, b2.reshape(1, D_H), W3)
    p3 = _sc_scatter(h3, pk0, pk1)
    h4 = _tc_mid(p3, h3, dinv, b3.reshape(1, D_H), W4)
    p4 = _sc_scatter(h4, pk0, pk1)
    u = _tc_mid(p4, h4, dinv, b4.reshape(1, D_H), None)
    p5 = _sc_scatter(u, pk0, pk1)
    W5p = jnp.pad(W5, ((0, 0), (0, CP - N_CLS)))
    b5p = jnp.pad(b5, (0, CP - N_CLS)).reshape(1, CP)
    Wlp = jnp.pad(Wl, ((0, CP - N_CLS), (0, CP - N_CLS)))
    blp = jnp.pad(bl, (0, CP - N_CLS)).reshape(1, CP)
    out = _tc_final(p5, u, dinv, W5p, b5p, Wlp, blp)
    return out[:, :N_CLS]


# speedup vs baseline: 1.0783x; 1.0783x over previous
import functools

import jax
import jax.numpy as jnp
from jax import lax
from jax.experimental import pallas as pl
from jax.experimental.pallas import tpu as pltpu
from jax.experimental.pallas import tpu_sc as plsc

N_NODES = 10000
N_EDGES = 320000
D_H = 128
N_CLS = 25
CP = 32

NC = 2
NS = 16
NW = NC * NS
K = 112
SLOW_CORE = 0
CH_SLOW = 48
CH_FAST = 132
E_SLOW = NS * CH_SLOW * K
E_PAD = NS * (CH_SLOW + CH_FAST) * K
NPAD = 10112
ZROWS = NPAD // NS

_MESH = plsc.VectorSubcoreMesh(core_axis_name="c", subcore_axis_name="s")


@functools.partial(
    pl.kernel,
    out_type=jax.ShapeDtypeStruct((NC, NPAD, D_H), jnp.float32),
    mesh=_MESH,
    compiler_params=pltpu.CompilerParams(needs_layout_passes=False),
    scratch_types=[
        pltpu.VMEM((CH_FAST, K), jnp.int32),
        pltpu.VMEM((8, K), jnp.int32),
        pltpu.VMEM((8, K), jnp.int32),
        pltpu.VMEM((K, D_H), jnp.float32),
        pltpu.VMEM((K, D_H), jnp.float32),
        pltpu.VMEM_SHARED((NPAD, D_H), jnp.float32),
        pltpu.SemaphoreType.DMA,
        pltpu.SemaphoreType.DMA,
    ],
)
def _sc_scatter(h_hbm, pk0_hbm, pk1_hbm, out_hbm, pk_v, usrc, udst,
                rows_a, rows_b, acc, sem_a, sem_b):
    c = lax.axis_index("c")
    s = lax.axis_index("s")

    @pl.when(c == SLOW_CORE)
    def _():
        pltpu.sync_copy(pk0_hbm.at[s], pk_v.at[pl.ds(0, CH_SLOW)])

    @pl.when(c != SLOW_CORE)
    def _():
        pltpu.sync_copy(pk1_hbm.at[s], pk_v)

    def unpack(j, r):
        for t in range(K // 16):
            w = pk_v[j, pl.ds(t * 16, 16)]
            usrc[r, pl.ds(t * 16, 16)] = w & 0xFFFF
            udst[r, pl.ds(t * 16, 16)] = lax.shift_right_logical(w, 16)

    def zbody(i, carry):
        rows_b[i // (D_H // 16), pl.ds((i % (D_H // 16)) * 16, 16)] = (
            jnp.zeros((16,), jnp.float32))
        return carry

    lax.fori_loop(0, K * D_H // 16, zbody, 0)

    unpack(0, 0)
    pltpu.async_copy(h_hbm.at[usrc.at[0]], rows_a, sem_a)
    for z in range(0, ZROWS, K):
        zn = min(K, ZROWS - z)
        pltpu.sync_copy(rows_b.at[pl.ds(0, zn)],
                        acc.at[pl.ds(s * ZROWS + z, zn)])
    plsc.subcore_barrier()

    def make_body(ch):
        def body(i, carry):
            j = 2 * i
            unpack(j + 1, 1)
            pltpu.async_copy(h_hbm.at[usrc.at[1]], rows_b, sem_b)
            pltpu.make_async_copy(h_hbm.at[usrc.at[0]], rows_a, sem_a).wait()
            pltpu.sync_copy(rows_a, acc.at[udst.at[0]], add=True)

            @pl.when(j + 2 < ch)
            def _():
                unpack(j + 2, 0)
                pltpu.async_copy(h_hbm.at[usrc.at[0]], rows_a, sem_a)

            pltpu.make_async_copy(h_hbm.at[usrc.at[1]], rows_b, sem_b).wait()
            pltpu.sync_copy(rows_b, acc.at[udst.at[1]], add=True)
            return carry

        return body

    @pl.when(c == SLOW_CORE)
    def _():
        lax.fori_loop(0, CH_SLOW // 2, make_body(CH_SLOW), 0)

    @pl.when(c != SLOW_CORE)
    def _():
        lax.fori_loop(0, CH_FAST // 2, make_body(CH_FAST), 0)

    plsc.subcore_barrier()
    pltpu.sync_copy(acc.at[pl.ds(s * ZROWS, ZROWS)],
                    out_hbm.at[c].at[pl.ds(s * ZROWS, ZROWS)])


@functools.partial(
    pl.kernel,
    out_type=jax.ShapeDtypeStruct((NW, 1, NPAD), jnp.float32),
    mesh=_MESH,
    compiler_params=pltpu.CompilerParams(needs_layout_passes=False),
    scratch_types=[
        pltpu.VMEM((CH_FAST, K), jnp.int32),
        pltpu.VMEM((NPAD,), jnp.float32),
        pltpu.SemaphoreType.DMA,
    ],
)
def _sc_degree(pk0_hbm, pk1_hbm, out_hbm, pk_v, acc_v, sem):
    c = lax.axis_index("c")
    s = lax.axis_index("s")
    wid = s * NC + c

    @pl.when(c == SLOW_CORE)
    def _():
        pltpu.sync_copy(pk0_hbm.at[s], pk_v.at[pl.ds(0, CH_SLOW)])

    @pl.when(c != SLOW_CORE)
    def _():
        pltpu.sync_copy(pk1_hbm.at[s], pk_v)

    def zbody(i, carry):
        acc_v[pl.ds(i * 16, 16)] = jnp.zeros((16,), jnp.float32)
        return carry

    lax.fori_loop(0, NPAD // 16, zbody, 0)
    ones = jnp.ones((16,), jnp.float32)
    g = K // 16

    def body(i, carry):
        idx = lax.shift_right_logical(
            pk_v[i // g, pl.ds((i % g) * 16, 16)], 16)
        plsc.addupdate_scatter(acc_v, [idx], ones)
        return carry

    @pl.when(c == SLOW_CORE)
    def _():
        lax.fori_loop(0, CH_SLOW * g, body, 0)

    @pl.when(c != SLOW_CORE)
    def _():
        lax.fori_loop(0, CH_FAST * g, body, 0)

    pltpu.sync_copy(acc_v, out_hbm.at[wid].at[0])


_BM = 1024
_GRID = (N_NODES + _BM - 1) // _BM
_T_DN = (((0,), (0,)), ((), ()))


def _tc_first(x, W, degp):
    def body(x_ref, w_ref, degp_ref, h_ref, dinv_ref):
        colsum = lax.dot_general(degp_ref[...], jnp.ones((NW, 1), jnp.float32),
                                 _T_DN, preferred_element_type=jnp.float32)
        dinv = lax.rsqrt(1.0 + colsum)
        dinv_ref[...] = dinv
        h_ref[...] = jnp.dot(x_ref[...], w_ref[...],
                             preferred_element_type=jnp.float32) * dinv

    return pl.pallas_call(
        body,
        grid=(_GRID,),
        in_specs=[
            pl.BlockSpec((_BM, D_H), lambda i: (i, 0)),
            pl.BlockSpec((D_H, D_H), lambda i: (0, 0)),
            pl.BlockSpec((NW, _BM), lambda i: (0, i)),
        ],
        out_specs=[
            pl.BlockSpec((_BM, D_H), lambda i: (i, 0)),
            pl.BlockSpec((_BM, 1), lambda i: (i, 0)),
        ],
        out_shape=[
            jax.ShapeDtypeStruct((N_NODES, D_H), jnp.float32),
            jax.ShapeDtypeStruct((N_NODES, 1), jnp.float32),
        ],
    )(x, W, degp)


def _tc_mid(p, hp, dinv, b, W, relu=False):
    def body(p_ref, hp_ref, dinv_ref, b_ref, *rest):
        o_ref = rest[-1]
        dinv = dinv_ref[...]
        t = (p_ref[0] + p_ref[1] + hp_ref[...]) * dinv + b_ref[...]
        if relu:
            t = jnp.maximum(t, 0.0)
        if W is not None:
            t = jnp.dot(t, rest[0][...], preferred_element_type=jnp.float32)
        o_ref[...] = t * dinv

    in_specs = [
        pl.BlockSpec((NC, _BM, D_H), lambda i: (0, i, 0)),
        pl.BlockSpec((_BM, D_H), lambda i: (i, 0)),
        pl.BlockSpec((_BM, 1), lambda i: (i, 0)),
        pl.BlockSpec((1, D_H), lambda i: (0, 0)),
    ]
    args = [p, hp, dinv, b]
    if W is not None:
        in_specs.append(pl.BlockSpec((D_H, D_H), lambda i: (0, 0)))
        args.append(W)
    return pl.pallas_call(
        body,
        grid=(_GRID,),
        in_specs=in_specs,
        out_specs=pl.BlockSpec((_BM, D_H), lambda i: (i, 0)),
        out_shape=jax.ShapeDtypeStruct((N_NODES, D_H), jnp.float32),
    )(*args)


def _tc_final(p, u, dinv, W5, b5, Wl, bl):
    def body(p_ref, u_ref, dinv_ref, w5_ref, b5_ref, wl_ref, bl_ref, o_ref):
        t = jnp.dot(p_ref[0] + p_ref[1] + u_ref[...], w5_ref[...],
                    preferred_element_type=jnp.float32)
        out5 = t * dinv_ref[...] + b5_ref[...]
        o_ref[...] = jnp.dot(out5, wl_ref[...],
                             preferred_element_type=jnp.float32) + bl_ref[...]

    return pl.pallas_call(
        body,
        grid=(_GRID,),
        in_specs=[
            pl.BlockSpec((NC, _BM, D_H), lambda i: (0, i, 0)),
            pl.BlockSpec((_BM, D_H), lambda i: (i, 0)),
            pl.BlockSpec((_BM, 1), lambda i: (i, 0)),
            pl.BlockSpec((D_H, CP), lambda i: (0, 0)),
            pl.BlockSpec((1, CP), lambda i: (0, 0)),
            pl.BlockSpec((CP, CP), lambda i: (0, 0)),
            pl.BlockSpec((1, CP), lambda i: (0, 0)),
        ],
        out_specs=pl.BlockSpec((_BM, CP), lambda i: (i, 0)),
        out_shape=jax.ShapeDtypeStruct((N_NODES, CP), jnp.float32),
    )(p, u, dinv, W5, b5, Wl, bl)


def kernel(x, edge_index, W1, b1, W2, b2, W3, b3, W4, b4, W5, b5, Wl, bl):
    src = edge_index[0].astype(jnp.int32)
    dst = edge_index[1].astype(jnp.int32)
    pad = E_PAD - N_EDGES
    src_p = jnp.concatenate([src, jnp.zeros((pad,), jnp.int32)])
    trash = N_NODES + (jnp.arange(pad, dtype=jnp.int32) % (NPAD - N_NODES))
    dst_p = jnp.concatenate([dst, trash])
    pk_all = src_p | (dst_p << 16)
    pk0 = pk_all[:E_SLOW].reshape(NS, CH_SLOW, K)
    pk1 = pk_all[E_SLOW:].reshape(NS, CH_FAST, K)
    degp = _sc_degree(pk0, pk1).reshape(NW, NPAD)
    h1, dinv = _tc_first(x, W1, degp)
    p1 = _sc_scatter(h1, pk0, pk1)
    h2 = _tc_mid(p1, h1, dinv, b1.reshape(1, D_H), W2, relu=True)
    p2 = _sc_scatter(h2, pk0, pk1)
    h3 = _tc_mid(p2, h2, dinv, b2.reshape(1, D_H), W3)
    p3 = _sc_scatter(h3, pk0, pk1)
    h4 = _tc_mid(p3, h3, dinv, b3.reshape(1, D_H), W4)
    p4 = _sc_scatter(h4, pk0, pk1)
    u = _tc_mid(p4, h4, dinv, b4.reshape(1, D_H), None)
    p5 = _sc_scatter(u, pk0, pk1)
    W5p = jnp.pad(W5, ((0, 0), (0, CP - N_CLS)))
    b5p = jnp.pad(b5, (0, CP - N_CLS)).reshape(1, CP)
    Wlp = jnp.pad(Wl, ((0, CP - N_CLS), (0, CP - N_CLS)))
    blp = jnp.pad(bl, (0, CP - N_CLS)).reshape(1, CP)
    out = _tc_final(p5, u, dinv, W5p, b5p, Wlp, blp)
    return out[:, :N_CLS]


# speedup vs baseline: 1.1051x; 1.0248x over previous
import functools

import jax
import jax.numpy as jnp
from jax import lax
from jax.experimental import pallas as pl
from jax.experimental.pallas import tpu as pltpu
from jax.experimental.pallas import tpu_sc as plsc

N_NODES = 10000
N_EDGES = 320000
D_H = 128
N_CLS = 25
CP = 32

NC = 2
NS = 16
NW = NC * NS
K = 112
SLOW_CORE = 0
CH_SLOW = 52
CH_FAST = 128
E_SLOW = NS * CH_SLOW * K
E_PAD = NS * (CH_SLOW + CH_FAST) * K
NPAD = 10112
ZROWS = NPAD // NS

_MESH = plsc.VectorSubcoreMesh(core_axis_name="c", subcore_axis_name="s")


@functools.partial(
    pl.kernel,
    out_type=jax.ShapeDtypeStruct((NC, NPAD, D_H), jnp.float32),
    mesh=_MESH,
    compiler_params=pltpu.CompilerParams(needs_layout_passes=False),
    scratch_types=[
        pltpu.VMEM((CH_FAST, K), jnp.int32),
        pltpu.VMEM((8, K), jnp.int32),
        pltpu.VMEM((8, K), jnp.int32),
        pltpu.VMEM((K, D_H), jnp.float32),
        pltpu.VMEM((K, D_H), jnp.float32),
        pltpu.VMEM_SHARED((NPAD, D_H), jnp.float32),
        pltpu.SemaphoreType.DMA,
        pltpu.SemaphoreType.DMA,
    ],
)
def _sc_scatter(h_hbm, pk0_hbm, pk1_hbm, out_hbm, pk_v, usrc, udst,
                rows_a, rows_b, acc, sem_a, sem_b):
    c = lax.axis_index("c")
    s = lax.axis_index("s")

    @pl.when(c == SLOW_CORE)
    def _():
        pltpu.sync_copy(pk0_hbm.at[s], pk_v.at[pl.ds(0, CH_SLOW)])

    @pl.when(c != SLOW_CORE)
    def _():
        pltpu.sync_copy(pk1_hbm.at[s], pk_v)

    def unpack(j, r):
        for t in range(K // 16):
            w = pk_v[j, pl.ds(t * 16, 16)]
            usrc[r, pl.ds(t * 16, 16)] = w & 0xFFFF
            udst[r, pl.ds(t * 16, 16)] = lax.shift_right_logical(w, 16)

    def zbody(i, carry):
        rows_b[i // (D_H // 16), pl.ds((i % (D_H // 16)) * 16, 16)] = (
            jnp.zeros((16,), jnp.float32))
        return carry

    lax.fori_loop(0, K * D_H // 16, zbody, 0)

    unpack(0, 0)
    pltpu.async_copy(h_hbm.at[usrc.at[0]], rows_a, sem_a)
    for z in range(0, ZROWS, K):
        zn = min(K, ZROWS - z)
        pltpu.sync_copy(rows_b.at[pl.ds(0, zn)],
                        acc.at[pl.ds(s * ZROWS + z, zn)])
    plsc.subcore_barrier()

    def make_body(ch):
        def body(i, carry):
            j = 2 * i
            unpack(j + 1, 1)
            pltpu.async_copy(h_hbm.at[usrc.at[1]], rows_b, sem_b)
            pltpu.make_async_copy(h_hbm.at[usrc.at[0]], rows_a, sem_a).wait()
            pltpu.sync_copy(rows_a, acc.at[udst.at[0]], add=True)

            @pl.when(j + 2 < ch)
            def _():
                unpack(j + 2, 0)
                pltpu.async_copy(h_hbm.at[usrc.at[0]], rows_a, sem_a)

            pltpu.make_async_copy(h_hbm.at[usrc.at[1]], rows_b, sem_b).wait()
            pltpu.sync_copy(rows_b, acc.at[udst.at[1]], add=True)
            return carry

        return body

    @pl.when(c == SLOW_CORE)
    def _():
        lax.fori_loop(0, CH_SLOW // 2, make_body(CH_SLOW), 0)

    @pl.when(c != SLOW_CORE)
    def _():
        lax.fori_loop(0, CH_FAST // 2, make_body(CH_FAST), 0)

    plsc.subcore_barrier()
    pltpu.sync_copy(acc.at[pl.ds(s * ZROWS, ZROWS)],
                    out_hbm.at[c].at[pl.ds(s * ZROWS, ZROWS)])


@functools.partial(
    pl.kernel,
    out_type=jax.ShapeDtypeStruct((NW, 1, NPAD), jnp.float32),
    mesh=_MESH,
    compiler_params=pltpu.CompilerParams(needs_layout_passes=False),
    scratch_types=[
        pltpu.VMEM((CH_FAST, K), jnp.int32),
        pltpu.VMEM((NPAD,), jnp.float32),
        pltpu.SemaphoreType.DMA,
    ],
)
def _sc_degree(pk0_hbm, pk1_hbm, out_hbm, pk_v, acc_v, sem):
    c = lax.axis_index("c")
    s = lax.axis_index("s")
    wid = s * NC + c

    @pl.when(c == SLOW_CORE)
    def _():
        pltpu.sync_copy(pk0_hbm.at[s], pk_v.at[pl.ds(0, CH_SLOW)])

    @pl.when(c != SLOW_CORE)
    def _():
        pltpu.sync_copy(pk1_hbm.at[s], pk_v)

    def zbody(i, carry):
        acc_v[pl.ds(i * 16, 16)] = jnp.zeros((16,), jnp.float32)
        return carry

    lax.fori_loop(0, NPAD // 16, zbody, 0)
    ones = jnp.ones((16,), jnp.float32)
    g = K // 16

    def body(i, carry):
        idx = lax.shift_right_logical(
            pk_v[i // g, pl.ds((i % g) * 16, 16)], 16)
        plsc.addupdate_scatter(acc_v, [idx], ones)
        return carry

    @pl.when(c == SLOW_CORE)
    def _():
        lax.fori_loop(0, CH_SLOW * g, body, 0)

    @pl.when(c != SLOW_CORE)
    def _():
        lax.fori_loop(0, CH_FAST * g, body, 0)

    pltpu.sync_copy(acc_v, out_hbm.at[wid].at[0])


_BM = 2048
_GRID = (N_NODES + _BM - 1) // _BM
_T_DN = (((0,), (0,)), ((), ()))


def _tc_first(x, W, degp):
    def body(x_ref, w_ref, degp_ref, h_ref, dinv_ref):
        colsum = lax.dot_general(degp_ref[...], jnp.ones((NW, 1), jnp.float32),
                                 _T_DN, preferred_element_type=jnp.float32)
        dinv = lax.rsqrt(1.0 + colsum)
        dinv_ref[...] = dinv
        h_ref[...] = jnp.dot(x_ref[...], w_ref[...],
                             preferred_element_type=jnp.float32) * dinv

    return pl.pallas_call(
        body,
        grid=(_GRID,),
        in_specs=[
            pl.BlockSpec((_BM, D_H), lambda i: (i, 0)),
            pl.BlockSpec((D_H, D_H), lambda i: (0, 0)),
            pl.BlockSpec((NW, _BM), lambda i: (0, i)),
        ],
        out_specs=[
            pl.BlockSpec((_BM, D_H), lambda i: (i, 0)),
            pl.BlockSpec((_BM, 1), lambda i: (i, 0)),
        ],
        out_shape=[
            jax.ShapeDtypeStruct((N_NODES, D_H), jnp.float32),
            jax.ShapeDtypeStruct((N_NODES, 1), jnp.float32),
        ],
    )(x, W, degp)


def _tc_mid(p, hp, dinv, b, W, relu=False):
    def body(p_ref, hp_ref, dinv_ref, b_ref, *rest):
        o_ref = rest[-1]
        dinv = dinv_ref[...]
        t = (p_ref[0] + p_ref[1] + hp_ref[...]) * dinv + b_ref[...]
        if relu:
            t = jnp.maximum(t, 0.0)
        if W is not None:
            t = jnp.dot(t, rest[0][...], preferred_element_type=jnp.float32)
        o_ref[...] = t * dinv

    in_specs = [
        pl.BlockSpec((NC, _BM, D_H), lambda i: (0, i, 0)),
        pl.BlockSpec((_BM, D_H), lambda i: (i, 0)),
        pl.BlockSpec((_BM, 1), lambda i: (i, 0)),
        pl.BlockSpec((1, D_H), lambda i: (0, 0)),
    ]
    args = [p, hp, dinv, b]
    if W is not None:
        in_specs.append(pl.BlockSpec((D_H, D_H), lambda i: (0, 0)))
        args.append(W)
    return pl.pallas_call(
        body,
        grid=(_GRID,),
        in_specs=in_specs,
        out_specs=pl.BlockSpec((_BM, D_H), lambda i: (i, 0)),
        out_shape=jax.ShapeDtypeStruct((N_NODES, D_H), jnp.float32),
    )(*args)


def _tc_final(p, u, dinv, W5, b5, Wl, bl):
    def body(p_ref, u_ref, dinv_ref, w5_ref, b5_ref, wl_ref, bl_ref, o_ref):
        t = jnp.dot(p_ref[0] + p_ref[1] + u_ref[...], w5_ref[...],
                    preferred_element_type=jnp.float32)
        out5 = t * dinv_ref[...] + b5_ref[...]
        o_ref[...] = jnp.dot(out5, wl_ref[...],
                             preferred_element_type=jnp.float32) + bl_ref[...]

    return pl.pallas_call(
        body,
        grid=(_GRID,),
        in_specs=[
            pl.BlockSpec((NC, _BM, D_H), lambda i: (0, i, 0)),
            pl.BlockSpec((_BM, D_H), lambda i: (i, 0)),
            pl.BlockSpec((_BM, 1), lambda i: (i, 0)),
            pl.BlockSpec((D_H, CP), lambda i: (0, 0)),
            pl.BlockSpec((1, CP), lambda i: (0, 0)),
            pl.BlockSpec((CP, CP), lambda i: (0, 0)),
            pl.BlockSpec((1, CP), lambda i: (0, 0)),
        ],
        out_specs=pl.BlockSpec((_BM, CP), lambda i: (i, 0)),
        out_shape=jax.ShapeDtypeStruct((N_NODES, CP), jnp.float32),
    )(p, u, dinv, W5, b5, Wl, bl)


def kernel(x, edge_index, W1, b1, W2, b2, W3, b3, W4, b4, W5, b5, Wl, bl):
    src = edge_index[0].astype(jnp.int32)
    dst = edge_index[1].astype(jnp.int32)
    pad = E_PAD - N_EDGES
    src_p = jnp.concatenate([src, jnp.zeros((pad,), jnp.int32)])
    trash = N_NODES + (jnp.arange(pad, dtype=jnp.int32) % (NPAD - N_NODES))
    dst_p = jnp.concatenate([dst, trash])
    pk_all = src_p | (dst_p << 16)
    pk0 = pk_all[:E_SLOW].reshape(NS, CH_SLOW, K)
    pk1 = pk_all[E_SLOW:].reshape(NS, CH_FAST, K)
    degp = _sc_degree(pk0, pk1).reshape(NW, NPAD)
    h1, dinv = _tc_first(x, W1, degp)
    p1 = _sc_scatter(h1, pk0, pk1)
    h2 = _tc_mid(p1, h1, dinv, b1.reshape(1, D_H), W2, relu=True)
    p2 = _sc_scatter(h2, pk0, pk1)
    h3 = _tc_mid(p2, h2, dinv, b2.reshape(1, D_H), W3)
    p3 = _sc_scatter(h3, pk0, pk1)
    h4 = _tc_mid(p3, h3, dinv, b3.reshape(1, D_H), W4)
    p4 = _sc_scatter(h4, pk0, pk1)
    u = _tc_mid(p4, h4, dinv, b4.reshape(1, D_H), None)
    p5 = _sc_scatter(u, pk0, pk1)
    W5p = jnp.pad(W5, ((0, 0), (0, CP - N_CLS)))
    b5p = jnp.pad(b5, (0, CP - N_CLS)).reshape(1, CP)
    Wlp = jnp.pad(Wl, ((0, CP - N_CLS), (0, CP - N_CLS)))
    blp = jnp.pad(bl, (0, CP - N_CLS)).reshape(1, CP)
    out = _tc_final(p5, u, dinv, W5p, b5p, Wlp, blp)
    return out[:, :N_CLS]
